# baseline (speedup 1.0000x reference)
import functools

import jax
import jax.numpy as jnp
from jax import lax
from jax.experimental import pallas as pl
from jax.experimental.pallas import tpu as pltpu
from jax.experimental.pallas import tpu_sc as plsc

V = 1_000_000
D = 16
BLK = 8000
GRID = V // BLK

B_TOTAL = 16384 * 50
NW = 32
PER_W = B_TOTAL // NW
CHUNK = 1024
NCHUNK = PER_W // CHUNK


def _score_body(e0, e1, e2, wref, bref, qref, s_ref):
    i = pl.program_id(0)

    @pl.when(i == 0)
    def _init():
        s_ref[0] = 0.0
        s_ref[1] = 0.0
        s_ref[2] = 0.0

    w = wref[...]
    bvec = bref[...]
    qvec = qref[...]
    for k, e in enumerate((e0, e1, e2)):
        h = jnp.tanh(jnp.dot(e[...], w, preferred_element_type=jnp.float32) + bvec)
        s_ref[k] += jnp.sum(h * qvec)


def _scores(e0, e1, e2, w, b2d, q2d):
    blk = pl.BlockSpec((BLK, D), lambda i: (i, 0))
    return pl.pallas_call(
        _score_body,
        grid=(GRID,),
        in_specs=[
            blk,
            blk,
            blk,
            pl.BlockSpec((D, D), lambda i: (0, 0)),
            pl.BlockSpec((1, D), lambda i: (0, 0)),
            pl.BlockSpec((1, D), lambda i: (0, 0)),
        ],
        out_specs=pl.BlockSpec(memory_space=pltpu.SMEM),
        out_shape=jax.ShapeDtypeStruct((3,), jnp.float32),
    )(e0, e1, e2, w, b2d, q2d)


def _gather_combine(e0, e1, e2, idx, scores_b):
    mesh = plsc.VectorSubcoreMesh(core_axis_name="c", subcore_axis_name="s")

    @functools.partial(
        pl.kernel,
        mesh=mesh,
        compiler_params=pltpu.CompilerParams(use_tc_tiling_on_sc=False),
        out_type=jax.ShapeDtypeStruct((B_TOTAL * D,), jnp.float32),
        scratch_types=[
            pltpu.VMEM((CHUNK,), jnp.int32),
            pltpu.VMEM((CHUNK, D), jnp.float32),
            pltpu.VMEM((CHUNK, D), jnp.float32),
            pltpu.VMEM((CHUNK, D), jnp.float32),
            pltpu.VMEM((CHUNK * D,), jnp.float32),
            pltpu.VMEM((3 * D,), jnp.float32),
            pltpu.SemaphoreType.DMA,
            pltpu.SemaphoreType.DMA,
            pltpu.SemaphoreType.DMA,
        ],
    )
    def k(e0_h, e1_h, e2_h, idx_h, sb_h, out_h,
          idx_v, r0, r1, r2, rflat, sv, sem0, sem1, sem2):
        wid = lax.axis_index("s") * 2 + lax.axis_index("c")
        pltpu.sync_copy(sb_h, sv)
        s0 = sv[pl.ds(0, D)] * (1.0 / V)
        s1 = sv[pl.ds(D, D)] * (1.0 / V)
        s2 = sv[pl.ds(2 * D, D)] * (1.0 / V)
        m = jnp.maximum(s0, jnp.maximum(s1, s2))
        x0 = jnp.exp(s0 - m)
        x1 = jnp.exp(s1 - m)
        x2 = jnp.exp(s2 - m)
        tot = x0 + x1 + x2
        b0 = x0 / tot
        b1 = x1 / tot
        b2 = x2 / tot

        def chunk(c, carry):
            base = wid * PER_W + c * CHUNK
            pltpu.sync_copy(idx_h.at[pl.ds(base, CHUNK)], idx_v)
            cp0 = pltpu.async_copy(e0_h.at[idx_v], r0, sem0)
            cp1 = pltpu.async_copy(e1_h.at[idx_v], r1, sem1)
            cp2 = pltpu.async_copy(e2_h.at[idx_v], r2, sem2)
            cp0.wait()
            cp1.wait()
            cp2.wait()

            def row(i, cc):
                rflat[pl.ds(i * D, D)] = r0[i, :] * b0 + r1[i, :] * b1 + r2[i, :] * b2
                return cc

            lax.fori_loop(0, CHUNK, row, 0, unroll=8)
            pltpu.sync_copy(rflat, out_h.at[pl.ds(base * D, CHUNK * D)])
            return carry

        lax.fori_loop(0, NCHUNK, chunk, 0)

    return k(e0, e1, e2, idx, scores_b)


def kernel(batch_ques, emb0, emb1, emb2, W, b, q):
    scores = _scores(emb0, emb1, emb2, W, b[None, :], q[None, :])
    scores_b = jnp.broadcast_to(scores[:, None], (3, D)).reshape(3 * D)
    idx = batch_ques.reshape(-1).astype(jnp.int32)
    out = _gather_combine(emb0, emb1, emb2, idx, scores_b)
    return out.reshape(batch_ques.shape + (D,))


# speedup vs baseline: 1.0119x; 1.0119x over previous
import functools

import jax
import jax.numpy as jnp
from jax import lax
from jax.experimental import pallas as pl
from jax.experimental.pallas import tpu as pltpu
from jax.experimental.pallas import tpu_sc as plsc

V = 1_000_000
D = 16
GROUPS = 8
VROWS = V // GROUPS
BLK = 5000
GRID = VROWS // BLK

B_TOTAL = 16384 * 50
NW = 32
PER_W = B_TOTAL // NW
CHUNK = 1024
NCHUNK = PER_W // CHUNK


def _score_body(e0, e1, e2, wref, bref, qref, s_ref):
    i = pl.program_id(0)

    @pl.when(i == 0)
    def _init():
        s_ref[0] = 0.0
        s_ref[1] = 0.0
        s_ref[2] = 0.0

    w = wref[...]
    bvec = bref[...]
    qvec = qref[...]
    for k, e in enumerate((e0, e1, e2)):
        h = jnp.tanh(jnp.dot(e[...], w, preferred_element_type=jnp.float32) + bvec)
        s_ref[k] += jnp.sum(h * qvec)


def _scores(e0v, e1v, e2v, wb, bb, qb):
    blk = pl.BlockSpec((BLK, 128), lambda i: (i, 0))
    return pl.pallas_call(
        _score_body,
        grid=(GRID,),
        in_specs=[
            blk,
            blk,
            blk,
            pl.BlockSpec((128, 128), lambda i: (0, 0)),
            pl.BlockSpec((1, 128), lambda i: (0, 0)),
            pl.BlockSpec((1, 128), lambda i: (0, 0)),
        ],
        out_specs=pl.BlockSpec(memory_space=pltpu.SMEM),
        out_shape=jax.ShapeDtypeStruct((3,), jnp.float32),
    )(e0v, e1v, e2v, wb, bb, qb)


def _gather_combine(e0, e1, e2, idx, scores_b):
    mesh = plsc.VectorSubcoreMesh(core_axis_name="c", subcore_axis_name="s")

    @functools.partial(
        pl.kernel,
        mesh=mesh,
        compiler_params=pltpu.CompilerParams(use_tc_tiling_on_sc=False),
        out_type=jax.ShapeDtypeStruct((B_TOTAL * D,), jnp.float32),
        scratch_types=[
            pltpu.VMEM((CHUNK,), jnp.int32),
            pltpu.VMEM((CHUNK, D), jnp.float32),
            pltpu.VMEM((CHUNK, D), jnp.float32),
            pltpu.VMEM((CHUNK, D), jnp.float32),
            pltpu.VMEM((CHUNK * D,), jnp.float32),
            pltpu.VMEM((3 * D,), jnp.float32),
            pltpu.SemaphoreType.DMA,
            pltpu.SemaphoreType.DMA,
            pltpu.SemaphoreType.DMA,
        ],
    )
    def k(e0_h, e1_h, e2_h, idx_h, sb_h, out_h,
          idx_v, r0, r1, r2, rflat, sv, sem0, sem1, sem2):
        wid = lax.axis_index("s") * 2 + lax.axis_index("c")
        pltpu.sync_copy(sb_h, sv)
        s0 = sv[pl.ds(0, D)] * (1.0 / V)
        s1 = sv[pl.ds(D, D)] * (1.0 / V)
        s2 = sv[pl.ds(2 * D, D)] * (1.0 / V)
        m = jnp.maximum(s0, jnp.maximum(s1, s2))
        x0 = jnp.exp(s0 - m)
        x1 = jnp.exp(s1 - m)
        x2 = jnp.exp(s2 - m)
        tot = x0 + x1 + x2
        b0 = x0 / tot
        b1 = x1 / tot
        b2 = x2 / tot

        def chunk(c, carry):
            base = wid * PER_W + c * CHUNK
            pltpu.sync_copy(idx_h.at[pl.ds(base, CHUNK)], idx_v)
            cp0 = pltpu.async_copy(e0_h.at[idx_v], r0, sem0)
            cp1 = pltpu.async_copy(e1_h.at[idx_v], r1, sem1)
            cp2 = pltpu.async_copy(e2_h.at[idx_v], r2, sem2)
            cp0.wait()
            cp1.wait()
            cp2.wait()

            def row(i, cc):
                rflat[pl.ds(i * D, D)] = r0[i, :] * b0 + r1[i, :] * b1 + r2[i, :] * b2
                return cc

            lax.fori_loop(0, CHUNK, row, 0, unroll=8)
            pltpu.sync_copy(rflat, out_h.at[pl.ds(base * D, CHUNK * D)])
            return carry

        lax.fori_loop(0, NCHUNK, chunk, 0)

    return k(e0, e1, e2, idx, scores_b)


def kernel(batch_ques, emb0, emb1, emb2, W, b, q):
    e0v = emb0.reshape(VROWS, GROUPS * D)
    e1v = emb1.reshape(VROWS, GROUPS * D)
    e2v = emb2.reshape(VROWS, GROUPS * D)
    wb = jnp.kron(jnp.eye(GROUPS, dtype=W.dtype), W)
    bb = jnp.tile(b, GROUPS)[None, :]
    qb = jnp.tile(q, GROUPS)[None, :]
    scores = _scores(e0v, e1v, e2v, wb, bb, qb)
    scores_b = jnp.broadcast_to(scores[:, None], (3, D)).reshape(3 * D)
    idx = batch_ques.reshape(-1).astype(jnp.int32)
    out = _gather_combine(emb0, emb1, emb2, idx, scores_b)
    return out.reshape(batch_ques.shape + (D,))


# speedup vs baseline: 1.4840x; 1.4666x over previous
import functools

import jax
import jax.numpy as jnp
from jax import lax
from jax.experimental import pallas as pl
from jax.experimental.pallas import tpu as pltpu
from jax.experimental.pallas import tpu_sc as plsc

V = 1_000_000
D = 16
GROUPS = 8
VROWS = V // GROUPS
BLK = 5000
GRID = VROWS // BLK

B_TOTAL = 16384 * 50
NW = 32
PER_W = B_TOTAL // NW
CHUNK = 1024
NCHUNK = PER_W // CHUNK


def _score_body(e0, e1, e2, wref, bref, qref, s_ref):
    i = pl.program_id(0)

    @pl.when(i == 0)
    def _init():
        s_ref[0] = 0.0
        s_ref[1] = 0.0
        s_ref[2] = 0.0

    w = wref[...]
    bvec = bref[...]
    qvec = qref[...]
    for k, e in enumerate((e0, e1, e2)):
        x = e[...].reshape(BLK, 128)
        h = jnp.tanh(jnp.dot(x, w, preferred_element_type=jnp.float32) + bvec)
        s_ref[k] += jnp.sum(h * qvec)


def _scores(e0f, e1f, e2f, wb, bb, qb):
    blk = pl.BlockSpec((BLK * 128,), lambda i: (i,))
    return pl.pallas_call(
        _score_body,
        grid=(GRID,),
        in_specs=[
            blk,
            blk,
            blk,
            pl.BlockSpec((128, 128), lambda i: (0, 0)),
            pl.BlockSpec((1, 128), lambda i: (0, 0)),
            pl.BlockSpec((1, 128), lambda i: (0, 0)),
        ],
        out_specs=pl.BlockSpec(memory_space=pltpu.SMEM),
        out_shape=jax.ShapeDtypeStruct((3,), jnp.float32),
    )(e0f, e1f, e2f, wb, bb, qb)


def _gather_combine(e0, e1, e2, idx, scores_b):
    mesh = plsc.VectorSubcoreMesh(core_axis_name="c", subcore_axis_name="s")

    @functools.partial(
        pl.kernel,
        mesh=mesh,
        compiler_params=pltpu.CompilerParams(use_tc_tiling_on_sc=False),
        out_type=jax.ShapeDtypeStruct((B_TOTAL * D,), jnp.float32),
        scratch_types=[
            pltpu.VMEM((CHUNK,), jnp.int32),
            pltpu.VMEM((CHUNK, D), jnp.float32),
            pltpu.VMEM((CHUNK, D), jnp.float32),
            pltpu.VMEM((CHUNK, D), jnp.float32),
            pltpu.VMEM((CHUNK * D,), jnp.float32),
            pltpu.VMEM((3 * D,), jnp.float32),
            pltpu.SemaphoreType.DMA,
            pltpu.SemaphoreType.DMA,
            pltpu.SemaphoreType.DMA,
        ],
    )
    def k(e0_h, e1_h, e2_h, idx_h, sb_h, out_h,
          idx_v, r0, r1, r2, rflat, sv, sem0, sem1, sem2):
        wid = lax.axis_index("s") * 2 + lax.axis_index("c")
        pltpu.sync_copy(sb_h, sv)
        s0 = sv[pl.ds(0, D)] * (1.0 / V)
        s1 = sv[pl.ds(D, D)] * (1.0 / V)
        s2 = sv[pl.ds(2 * D, D)] * (1.0 / V)
        m = jnp.maximum(s0, jnp.maximum(s1, s2))
        x0 = jnp.exp(s0 - m)
        x1 = jnp.exp(s1 - m)
        x2 = jnp.exp(s2 - m)
        tot = x0 + x1 + x2
        b0 = x0 / tot
        b1 = x1 / tot
        b2 = x2 / tot

        def chunk(c, carry):
            base = wid * PER_W + c * CHUNK
            pltpu.sync_copy(idx_h.at[pl.ds(base, CHUNK)], idx_v)
            cp0 = pltpu.async_copy(e0_h.at[idx_v], r0, sem0)
            cp1 = pltpu.async_copy(e1_h.at[idx_v], r1, sem1)
            cp2 = pltpu.async_copy(e2_h.at[idx_v], r2, sem2)
            cp0.wait()
            cp1.wait()
            cp2.wait()

            def row(i, cc):
                rflat[pl.ds(i * D, D)] = r0[i, :] * b0 + r1[i, :] * b1 + r2[i, :] * b2
                return cc

            lax.fori_loop(0, CHUNK, row, 0, unroll=8)
            pltpu.sync_copy(rflat, out_h.at[pl.ds(base * D, CHUNK * D)])
            return carry

        lax.fori_loop(0, NCHUNK, chunk, 0)

    return k(e0, e1, e2, idx, scores_b)


def kernel(batch_ques, emb0, emb1, emb2, W, b, q):
    wb = jnp.kron(jnp.eye(GROUPS, dtype=W.dtype), W)
    bb = jnp.tile(b, GROUPS)[None, :]
    qb = jnp.tile(q, GROUPS)[None, :]
    scores = _scores(emb0.reshape(-1), emb1.reshape(-1), emb2.reshape(-1),
                     wb, bb, qb)
    scores_b = jnp.broadcast_to(scores[:, None], (3, D)).reshape(3 * D)
    idx = batch_ques.reshape(-1).astype(jnp.int32)
    out = _gather_combine(emb0, emb1, emb2, idx, scores_b)
    return out.reshape(batch_ques.shape + (D,))
